# baseline (speedup 1.0000x reference)
import functools

import jax
import jax.numpy as jnp
from jax import lax
from jax.experimental import pallas as pl
from jax.experimental.pallas import tpu as pltpu
from jax.experimental.pallas import tpu_sc as plsc

NUM_NODES = 1000000
BATCH = 16384
DEFAULT_DT = 1.0

NUM_WORKERS = 16
EV_PER_W = BATCH // NUM_WORKERS
GCHUNK = 128
R_OWN = 62504
LAST_OWN = NUM_NODES - (NUM_WORKERS - 1) * R_OWN


def _body(ids_hbm, ts_hbm, last_hbm, dt_hbm, out_hbm,
          allids_v, allts_v, glast_v, gdt_v, own_v, sem, sem_own, sem_out):
    wid = lax.axis_index("s")
    nbase = wid * R_OWN
    ebase = wid * EV_PER_W

    @pl.when(wid < NUM_WORKERS - 1)
    def _():
        pltpu.async_copy(last_hbm.at[pl.ds(nbase, R_OWN)], own_v, sem_own)

    @pl.when(wid == NUM_WORKERS - 1)
    def _():
        pltpu.async_copy(last_hbm.at[pl.ds(nbase, LAST_OWN)],
                         own_v.at[pl.ds(0, LAST_OWN)], sem_own)

    c_ts = pltpu.async_copy(ts_hbm, allts_v, sem)
    pltpu.sync_copy(ids_hbm, allids_v)
    gathers = [
        pltpu.async_copy(
            last_hbm.at[allids_v.at[pl.ds(ebase + j * GCHUNK, GCHUNK)]],
            glast_v.at[pl.ds(j * GCHUNK, GCHUNK)],
            sem,
        )
        for j in range(EV_PER_W // GCHUNK)
    ]
    c_ts.wait()

    nsize = jnp.minimum(nbase + R_OWN, NUM_NODES) - nbase
    nsize_u = plsc.bitcast(jnp.broadcast_to(nsize, (16,)), jnp.uint32)

    @pl.when(wid < NUM_WORKERS - 1)
    def _():
        pltpu.make_async_copy(last_hbm.at[pl.ds(nbase, R_OWN)], own_v,
                              sem_own).wait()

    @pl.when(wid == NUM_WORKERS - 1)
    def _():
        pltpu.make_async_copy(last_hbm.at[pl.ds(nbase, LAST_OWN)],
                              own_v.at[pl.ds(0, LAST_OWN)], sem_own).wait()

    for i in range(BATCH // 16):
        off = i * 16
        loc = allids_v[pl.ds(off, 16)] - nbase
        m = plsc.bitcast(loc, jnp.uint32) < nsize_u
        tsv = allts_v[pl.ds(off, 16)]
        plsc.store_scatter(own_v, [loc], tsv, mask=m)

    @pl.when(wid < NUM_WORKERS - 1)
    def _():
        pltpu.async_copy(own_v, out_hbm.at[pl.ds(nbase, R_OWN)], sem_out)

    @pl.when(wid == NUM_WORKERS - 1)
    def _():
        pltpu.async_copy(own_v.at[pl.ds(0, LAST_OWN)],
                         out_hbm.at[pl.ds(nbase, LAST_OWN)], sem_out)

    for c in gathers:
        c.wait()
    one = jnp.full((16,), DEFAULT_DT, jnp.float32)
    zero = jnp.zeros((16,), jnp.float32)
    for i in range(EV_PER_W // 16):
        lastv = glast_v[pl.ds(i * 16, 16)]
        tsv = allts_v[pl.ds(ebase + i * 16, 16)]
        dtv = jnp.where(lastv >= 0.0, tsv - lastv, one)
        gdt_v[pl.ds(i * 16, 16)] = jnp.maximum(dtv, zero)
    pltpu.sync_copy(gdt_v, dt_hbm.at[pl.ds(ebase, EV_PER_W)])

    @pl.when(wid < NUM_WORKERS - 1)
    def _():
        pltpu.make_async_copy(own_v, out_hbm.at[pl.ds(nbase, R_OWN)],
                              sem_out).wait()

    @pl.when(wid == NUM_WORKERS - 1)
    def _():
        pltpu.make_async_copy(own_v.at[pl.ds(0, LAST_OWN)],
                              out_hbm.at[pl.ds(nbase, LAST_OWN)],
                              sem_out).wait()


_recency = functools.partial(
    pl.kernel,
    out_type=(
        jax.ShapeDtypeStruct((BATCH,), jnp.float32),
        jax.ShapeDtypeStruct((NUM_NODES,), jnp.float32),
    ),
    mesh=plsc.VectorSubcoreMesh(core_axis_name="c", subcore_axis_name="s",
                                num_cores=1),
    compiler_params=pltpu.CompilerParams(needs_layout_passes=False),
    scratch_types=[
        pltpu.VMEM((BATCH,), jnp.int32),
        pltpu.VMEM((BATCH,), jnp.float32),
        pltpu.VMEM((EV_PER_W,), jnp.float32),
        pltpu.VMEM((EV_PER_W,), jnp.float32),
        pltpu.VMEM((R_OWN,), jnp.float32),
        pltpu.SemaphoreType.DMA,
        pltpu.SemaphoreType.DMA,
        pltpu.SemaphoreType.DMA,
    ],
)(_body)


def kernel(node_ids, ts, last_src_ts):
    ids = node_ids.astype(jnp.int32)
    dt, new_last = _recency(ids, ts, last_src_ts)
    return dt, new_last


# speedup vs baseline: 1.2957x; 1.2957x over previous
import functools

import jax
import jax.numpy as jnp
from jax import lax
from jax.experimental import pallas as pl
from jax.experimental.pallas import tpu as pltpu
from jax.experimental.pallas import tpu_sc as plsc

NUM_NODES = 1000000
BATCH = 16384
DEFAULT_DT = 1.0

NUM_WORKERS = 16
EV_PER_W = BATCH // NUM_WORKERS
GCHUNK = 128
R_OWN = 62504
LAST_OWN = NUM_NODES - (NUM_WORKERS - 1) * R_OWN


def _body(ids_hbm, ts_hbm, last_hbm, dt_hbm, out_hbm,
          allids_v, allts_v, glast_v, gdt_v, own_v, sem, sem_own, sem_out):
    wid = lax.axis_index("s")
    nbase = wid * R_OWN
    ebase = wid * EV_PER_W

    @pl.when(wid < NUM_WORKERS - 1)
    def _():
        pltpu.async_copy(last_hbm.at[pl.ds(nbase, R_OWN)], own_v, sem_own)

    @pl.when(wid == NUM_WORKERS - 1)
    def _():
        pltpu.async_copy(last_hbm.at[pl.ds(nbase, LAST_OWN)],
                         own_v.at[pl.ds(0, LAST_OWN)], sem_own)

    c_ts = pltpu.async_copy(ts_hbm, allts_v, sem)
    pltpu.sync_copy(ids_hbm, allids_v)
    gathers = [
        pltpu.async_copy(
            last_hbm.at[allids_v.at[pl.ds(ebase + j * GCHUNK, GCHUNK)]],
            glast_v.at[pl.ds(j * GCHUNK, GCHUNK)],
            sem,
        )
        for j in range(EV_PER_W // GCHUNK)
    ]
    c_ts.wait()

    nsize = jnp.minimum(nbase + R_OWN, NUM_NODES) - nbase
    nsize_u = plsc.bitcast(jnp.broadcast_to(nsize, (16,)), jnp.uint32)

    @pl.when(wid < NUM_WORKERS - 1)
    def _():
        pltpu.make_async_copy(last_hbm.at[pl.ds(nbase, R_OWN)], own_v,
                              sem_own).wait()

    @pl.when(wid == NUM_WORKERS - 1)
    def _():
        pltpu.make_async_copy(last_hbm.at[pl.ds(nbase, LAST_OWN)],
                              own_v.at[pl.ds(0, LAST_OWN)], sem_own).wait()

    def sbody(i, carry):
        off = i * 16
        loc = allids_v[pl.ds(off, 16)] - nbase
        m = plsc.bitcast(loc, jnp.uint32) < nsize_u
        tsv = allts_v[pl.ds(off, 16)]
        plsc.store_scatter(own_v, [loc], tsv, mask=m)
        return carry

    lax.fori_loop(0, BATCH // 16, sbody, 0, unroll=16)

    @pl.when(wid < NUM_WORKERS - 1)
    def _():
        pltpu.async_copy(own_v, out_hbm.at[pl.ds(nbase, R_OWN)], sem_out)

    @pl.when(wid == NUM_WORKERS - 1)
    def _():
        pltpu.async_copy(own_v.at[pl.ds(0, LAST_OWN)],
                         out_hbm.at[pl.ds(nbase, LAST_OWN)], sem_out)

    for c in gathers:
        c.wait()
    one = jnp.full((16,), DEFAULT_DT, jnp.float32)
    zero = jnp.zeros((16,), jnp.float32)
    for i in range(EV_PER_W // 16):
        lastv = glast_v[pl.ds(i * 16, 16)]
        tsv = allts_v[pl.ds(ebase + i * 16, 16)]
        dtv = jnp.where(lastv >= 0.0, tsv - lastv, one)
        gdt_v[pl.ds(i * 16, 16)] = jnp.maximum(dtv, zero)
    pltpu.sync_copy(gdt_v, dt_hbm.at[pl.ds(ebase, EV_PER_W)])

    @pl.when(wid < NUM_WORKERS - 1)
    def _():
        pltpu.make_async_copy(own_v, out_hbm.at[pl.ds(nbase, R_OWN)],
                              sem_out).wait()

    @pl.when(wid == NUM_WORKERS - 1)
    def _():
        pltpu.make_async_copy(own_v.at[pl.ds(0, LAST_OWN)],
                              out_hbm.at[pl.ds(nbase, LAST_OWN)],
                              sem_out).wait()


_recency = functools.partial(
    pl.kernel,
    out_type=(
        jax.ShapeDtypeStruct((BATCH,), jnp.float32),
        jax.ShapeDtypeStruct((NUM_NODES,), jnp.float32),
    ),
    mesh=plsc.VectorSubcoreMesh(core_axis_name="c", subcore_axis_name="s",
                                num_cores=1),
    compiler_params=pltpu.CompilerParams(needs_layout_passes=False),
    scratch_types=[
        pltpu.VMEM((BATCH,), jnp.int32),
        pltpu.VMEM((BATCH,), jnp.float32),
        pltpu.VMEM((EV_PER_W,), jnp.float32),
        pltpu.VMEM((EV_PER_W,), jnp.float32),
        pltpu.VMEM((R_OWN,), jnp.float32),
        pltpu.SemaphoreType.DMA,
        pltpu.SemaphoreType.DMA,
        pltpu.SemaphoreType.DMA,
    ],
)(_body)


def kernel(node_ids, ts, last_src_ts):
    ids = node_ids.astype(jnp.int32)
    dt, new_last = _recency(ids, ts, last_src_ts)
    return dt, new_last


# speedup vs baseline: 1.3394x; 1.0337x over previous
import functools

import jax
import jax.numpy as jnp
from jax import lax
from jax.experimental import pallas as pl
from jax.experimental.pallas import tpu as pltpu
from jax.experimental.pallas import tpu_sc as plsc

NUM_NODES = 1000000
BATCH = 16384
DEFAULT_DT = 1.0

NUM_WORKERS = 16
EV_PER_W = BATCH // NUM_WORKERS
GCHUNK = 128
R_OWN = 62504
LAST_OWN = NUM_NODES - (NUM_WORKERS - 1) * R_OWN


def _body(ids_hbm, ts_hbm, last_hbm, dt_hbm, out_hbm,
          allids_v, allts_v, glast_v, gdt_v, own_v, sem, sem_own, sem_out):
    wid = lax.axis_index("s")
    nbase = wid * R_OWN
    ebase = wid * EV_PER_W

    @pl.when(wid < NUM_WORKERS - 1)
    def _():
        pltpu.async_copy(last_hbm.at[pl.ds(nbase, R_OWN)],
                         own_v.at[pl.ds(8, R_OWN)], sem_own)

    @pl.when(wid == NUM_WORKERS - 1)
    def _():
        pltpu.async_copy(last_hbm.at[pl.ds(nbase, LAST_OWN)],
                         own_v.at[pl.ds(8, LAST_OWN)], sem_own)

    c_ts = pltpu.async_copy(ts_hbm, allts_v, sem)
    pltpu.sync_copy(ids_hbm, allids_v)
    gathers = [
        pltpu.async_copy(
            last_hbm.at[allids_v.at[pl.ds(ebase + j * GCHUNK, GCHUNK)]],
            glast_v.at[pl.ds(j * GCHUNK, GCHUNK)],
            sem,
        )
        for j in range(EV_PER_W // GCHUNK)
    ]
    c_ts.wait()

    nsize = jnp.minimum(nbase + R_OWN, NUM_NODES) - nbase
    cap_u = plsc.bitcast(jnp.broadcast_to(nsize + 8, (16,))
                         + lax.iota(jnp.int32, 16), jnp.uint32)
    base_m8 = jnp.broadcast_to(nbase - 8, (16,))

    @pl.when(wid < NUM_WORKERS - 1)
    def _():
        pltpu.make_async_copy(last_hbm.at[pl.ds(nbase, R_OWN)],
                              own_v.at[pl.ds(8, R_OWN)], sem_own).wait()

    @pl.when(wid == NUM_WORKERS - 1)
    def _():
        pltpu.make_async_copy(last_hbm.at[pl.ds(nbase, LAST_OWN)],
                              own_v.at[pl.ds(8, LAST_OWN)], sem_own).wait()

    def sbody(i, carry):
        off = i * 16
        d_u = plsc.bitcast(allids_v[pl.ds(off, 16)] - base_m8, jnp.uint32)
        loc = plsc.bitcast(jnp.minimum(d_u, cap_u), jnp.int32)
        tsv = allts_v[pl.ds(off, 16)]
        plsc.store_scatter(own_v, [loc], tsv)
        return carry

    lax.fori_loop(0, BATCH // 16, sbody, 0, unroll=16)

    @pl.when(wid < NUM_WORKERS - 1)
    def _():
        pltpu.async_copy(own_v.at[pl.ds(8, R_OWN)],
                         out_hbm.at[pl.ds(nbase, R_OWN)], sem_out)

    @pl.when(wid == NUM_WORKERS - 1)
    def _():
        pltpu.async_copy(own_v.at[pl.ds(8, LAST_OWN)],
                         out_hbm.at[pl.ds(nbase, LAST_OWN)], sem_out)

    for c in gathers:
        c.wait()
    one = jnp.full((16,), DEFAULT_DT, jnp.float32)
    zero = jnp.zeros((16,), jnp.float32)

    def dbody(i, carry):
        lastv = glast_v[pl.ds(i * 16, 16)]
        tsv = allts_v[pl.ds(ebase + i * 16, 16)]
        dtv = jnp.where(lastv >= 0.0, tsv - lastv, one)
        gdt_v[pl.ds(i * 16, 16)] = jnp.maximum(dtv, zero)
        return carry

    lax.fori_loop(0, EV_PER_W // 16, dbody, 0, unroll=8)
    pltpu.sync_copy(gdt_v, dt_hbm.at[pl.ds(ebase, EV_PER_W)])

    @pl.when(wid < NUM_WORKERS - 1)
    def _():
        pltpu.make_async_copy(own_v.at[pl.ds(8, R_OWN)],
                              out_hbm.at[pl.ds(nbase, R_OWN)], sem_out).wait()

    @pl.when(wid == NUM_WORKERS - 1)
    def _():
        pltpu.make_async_copy(own_v.at[pl.ds(8, LAST_OWN)],
                              out_hbm.at[pl.ds(nbase, LAST_OWN)],
                              sem_out).wait()


_recency = functools.partial(
    pl.kernel,
    out_type=(
        jax.ShapeDtypeStruct((BATCH,), jnp.float32),
        jax.ShapeDtypeStruct((NUM_NODES,), jnp.float32),
    ),
    mesh=plsc.VectorSubcoreMesh(core_axis_name="c", subcore_axis_name="s",
                                num_cores=1),
    compiler_params=pltpu.CompilerParams(needs_layout_passes=False),
    scratch_types=[
        pltpu.VMEM((BATCH,), jnp.int32),
        pltpu.VMEM((BATCH,), jnp.float32),
        pltpu.VMEM((EV_PER_W,), jnp.float32),
        pltpu.VMEM((EV_PER_W,), jnp.float32),
        pltpu.VMEM((R_OWN + 32,), jnp.float32),
        pltpu.SemaphoreType.DMA,
        pltpu.SemaphoreType.DMA,
        pltpu.SemaphoreType.DMA,
    ],
)(_body)


def kernel(node_ids, ts, last_src_ts):
    ids = node_ids.astype(jnp.int32)
    dt, new_last = _recency(ids, ts, last_src_ts)
    return dt, new_last


# speedup vs baseline: 1.3401x; 1.0005x over previous
import functools

import jax
import jax.numpy as jnp
from jax import lax
from jax.experimental import pallas as pl
from jax.experimental.pallas import tpu as pltpu
from jax.experimental.pallas import tpu_sc as plsc

NUM_NODES = 1000000
BATCH = 16384
DEFAULT_DT = 1.0

NUM_WORKERS = 16
EV_PER_W = BATCH // NUM_WORKERS
GCHUNK = 128
R_OWN = 62504
LAST_OWN = NUM_NODES - (NUM_WORKERS - 1) * R_OWN


def _body(ids_hbm, ts_hbm, last_hbm, dt_hbm, out_hbm,
          allids_v, allts_v, glast_v, gdt_v, own_v, sem, sem_own, sem_out):
    wid = lax.axis_index("s")
    nbase = wid * R_OWN
    ebase = wid * EV_PER_W

    @pl.when(wid < NUM_WORKERS - 1)
    def _():
        pltpu.async_copy(last_hbm.at[pl.ds(nbase, R_OWN)],
                         own_v.at[pl.ds(8, R_OWN)], sem_own)

    @pl.when(wid == NUM_WORKERS - 1)
    def _():
        pltpu.async_copy(last_hbm.at[pl.ds(nbase, LAST_OWN)],
                         own_v.at[pl.ds(8, LAST_OWN)], sem_own)

    c_ts = pltpu.async_copy(ts_hbm, allts_v, sem)
    pltpu.sync_copy(ids_hbm, allids_v)
    gathers = [
        pltpu.async_copy(
            last_hbm.at[allids_v.at[pl.ds(ebase + j * GCHUNK, GCHUNK)]],
            glast_v.at[pl.ds(j * GCHUNK, GCHUNK)],
            sem,
        )
        for j in range(EV_PER_W // GCHUNK)
    ]
    c_ts.wait()

    nsize = jnp.minimum(nbase + R_OWN, NUM_NODES) - nbase
    cap_u = plsc.bitcast(jnp.broadcast_to(nsize + 8, (16,))
                         + lax.iota(jnp.int32, 16), jnp.uint32)
    base_m8 = jnp.broadcast_to(nbase - 8, (16,))

    @pl.when(wid < NUM_WORKERS - 1)
    def _():
        pltpu.make_async_copy(last_hbm.at[pl.ds(nbase, R_OWN)],
                              own_v.at[pl.ds(8, R_OWN)], sem_own).wait()

    @pl.when(wid == NUM_WORKERS - 1)
    def _():
        pltpu.make_async_copy(last_hbm.at[pl.ds(nbase, LAST_OWN)],
                              own_v.at[pl.ds(8, LAST_OWN)], sem_own).wait()

    def sbody(i, carry):
        off = i * 16
        d_u = plsc.bitcast(allids_v[pl.ds(off, 16)] - base_m8, jnp.uint32)
        loc = plsc.bitcast(jnp.minimum(d_u, cap_u), jnp.int32)
        tsv = allts_v[pl.ds(off, 16)]
        plsc.store_scatter(own_v, [loc], tsv)
        return carry

    lax.fori_loop(0, BATCH // 16, sbody, 0, unroll=4)

    @pl.when(wid < NUM_WORKERS - 1)
    def _():
        pltpu.async_copy(own_v.at[pl.ds(8, R_OWN)],
                         out_hbm.at[pl.ds(nbase, R_OWN)], sem_out)

    @pl.when(wid == NUM_WORKERS - 1)
    def _():
        pltpu.async_copy(own_v.at[pl.ds(8, LAST_OWN)],
                         out_hbm.at[pl.ds(nbase, LAST_OWN)], sem_out)

    for c in gathers:
        c.wait()
    one = jnp.full((16,), DEFAULT_DT, jnp.float32)
    zero = jnp.zeros((16,), jnp.float32)

    def dbody(i, carry):
        lastv = glast_v[pl.ds(i * 16, 16)]
        tsv = allts_v[pl.ds(ebase + i * 16, 16)]
        dtv = jnp.where(lastv >= 0.0, tsv - lastv, one)
        gdt_v[pl.ds(i * 16, 16)] = jnp.maximum(dtv, zero)
        return carry

    lax.fori_loop(0, EV_PER_W // 16, dbody, 0, unroll=4)
    pltpu.sync_copy(gdt_v, dt_hbm.at[pl.ds(ebase, EV_PER_W)])

    @pl.when(wid < NUM_WORKERS - 1)
    def _():
        pltpu.make_async_copy(own_v.at[pl.ds(8, R_OWN)],
                              out_hbm.at[pl.ds(nbase, R_OWN)], sem_out).wait()

    @pl.when(wid == NUM_WORKERS - 1)
    def _():
        pltpu.make_async_copy(own_v.at[pl.ds(8, LAST_OWN)],
                              out_hbm.at[pl.ds(nbase, LAST_OWN)],
                              sem_out).wait()


_recency = functools.partial(
    pl.kernel,
    out_type=(
        jax.ShapeDtypeStruct((BATCH,), jnp.float32),
        jax.ShapeDtypeStruct((NUM_NODES,), jnp.float32),
    ),
    mesh=plsc.VectorSubcoreMesh(core_axis_name="c", subcore_axis_name="s",
                                num_cores=1),
    compiler_params=pltpu.CompilerParams(needs_layout_passes=False),
    scratch_types=[
        pltpu.VMEM((BATCH,), jnp.int32),
        pltpu.VMEM((BATCH,), jnp.float32),
        pltpu.VMEM((EV_PER_W,), jnp.float32),
        pltpu.VMEM((EV_PER_W,), jnp.float32),
        pltpu.VMEM((R_OWN + 32,), jnp.float32),
        pltpu.SemaphoreType.DMA,
        pltpu.SemaphoreType.DMA,
        pltpu.SemaphoreType.DMA,
    ],
)(_body)


def kernel(node_ids, ts, last_src_ts):
    ids = node_ids.astype(jnp.int32)
    dt, new_last = _recency(ids, ts, last_src_ts)
    return dt, new_last


# speedup vs baseline: 1.3432x; 1.0023x over previous
import functools

import jax
import jax.numpy as jnp
from jax import lax
from jax.experimental import pallas as pl
from jax.experimental.pallas import tpu as pltpu
from jax.experimental.pallas import tpu_sc as plsc

NUM_NODES = 1000000
BATCH = 16384
DEFAULT_DT = 1.0

NUM_WORKERS = 16
EV_PER_W = BATCH // NUM_WORKERS
GCHUNK = 128
R_OWN = 62504
LAST_OWN = NUM_NODES - (NUM_WORKERS - 1) * R_OWN


def _body(ids_hbm, ts_hbm, last_hbm, dt_hbm, out_hbm,
          allids_v, allts_v, glast_v, gdt_v, own_v, sem, sem_own, sem_out):
    wid = lax.axis_index("s")
    nbase = wid * R_OWN
    ebase = wid * EV_PER_W

    @pl.when(wid < NUM_WORKERS - 1)
    def _():
        pltpu.async_copy(last_hbm.at[pl.ds(nbase, R_OWN)],
                         own_v.at[pl.ds(8, R_OWN)], sem_own)

    @pl.when(wid == NUM_WORKERS - 1)
    def _():
        pltpu.async_copy(last_hbm.at[pl.ds(nbase, LAST_OWN)],
                         own_v.at[pl.ds(8, LAST_OWN)], sem_own)

    c_ts = pltpu.async_copy(ts_hbm, allts_v, sem)
    pltpu.sync_copy(ids_hbm, allids_v)
    gathers = [
        pltpu.async_copy(
            last_hbm.at[allids_v.at[pl.ds(ebase + j * GCHUNK, GCHUNK)]],
            glast_v.at[pl.ds(j * GCHUNK, GCHUNK)],
            sem,
        )
        for j in range(EV_PER_W // GCHUNK)
    ]
    c_ts.wait()

    nsize = jnp.minimum(nbase + R_OWN, NUM_NODES) - nbase
    cap_u = plsc.bitcast(jnp.broadcast_to(nsize + 8, (16,))
                         + lax.iota(jnp.int32, 16), jnp.uint32)
    base_m8 = jnp.broadcast_to(nbase - 8, (16,))

    @pl.when(wid < NUM_WORKERS - 1)
    def _():
        pltpu.make_async_copy(last_hbm.at[pl.ds(nbase, R_OWN)],
                              own_v.at[pl.ds(8, R_OWN)], sem_own).wait()

    @pl.when(wid == NUM_WORKERS - 1)
    def _():
        pltpu.make_async_copy(last_hbm.at[pl.ds(nbase, LAST_OWN)],
                              own_v.at[pl.ds(8, LAST_OWN)], sem_own).wait()

    def sbody(i, carry):
        off = i * 16
        d_u = plsc.bitcast(allids_v[pl.ds(off, 16)] - base_m8, jnp.uint32)
        loc = plsc.bitcast(jnp.minimum(d_u, cap_u), jnp.int32)
        tsv = allts_v[pl.ds(off, 16)]
        plsc.store_scatter(own_v, [loc], tsv)
        return carry

    lax.fori_loop(0, BATCH // 16, sbody, 0, unroll=8)

    @pl.when(wid < NUM_WORKERS - 1)
    def _():
        pltpu.async_copy(own_v.at[pl.ds(8, R_OWN)],
                         out_hbm.at[pl.ds(nbase, R_OWN)], sem_out)

    @pl.when(wid == NUM_WORKERS - 1)
    def _():
        pltpu.async_copy(own_v.at[pl.ds(8, LAST_OWN)],
                         out_hbm.at[pl.ds(nbase, LAST_OWN)], sem_out)

    for c in gathers:
        c.wait()
    one = jnp.full((16,), DEFAULT_DT, jnp.float32)
    zero = jnp.zeros((16,), jnp.float32)

    def dbody(i, carry):
        lastv = glast_v[pl.ds(i * 16, 16)]
        tsv = allts_v[pl.ds(ebase + i * 16, 16)]
        dtv = jnp.where(lastv >= 0.0, tsv - lastv, one)
        gdt_v[pl.ds(i * 16, 16)] = jnp.maximum(dtv, zero)
        return carry

    lax.fori_loop(0, EV_PER_W // 16, dbody, 0, unroll=8)
    pltpu.sync_copy(gdt_v, dt_hbm.at[pl.ds(ebase, EV_PER_W)])

    @pl.when(wid < NUM_WORKERS - 1)
    def _():
        pltpu.make_async_copy(own_v.at[pl.ds(8, R_OWN)],
                              out_hbm.at[pl.ds(nbase, R_OWN)], sem_out).wait()

    @pl.when(wid == NUM_WORKERS - 1)
    def _():
        pltpu.make_async_copy(own_v.at[pl.ds(8, LAST_OWN)],
                              out_hbm.at[pl.ds(nbase, LAST_OWN)],
                              sem_out).wait()


_recency = functools.partial(
    pl.kernel,
    out_type=(
        jax.ShapeDtypeStruct((BATCH,), jnp.float32),
        jax.ShapeDtypeStruct((NUM_NODES,), jnp.float32),
    ),
    mesh=plsc.VectorSubcoreMesh(core_axis_name="c", subcore_axis_name="s",
                                num_cores=1),
    compiler_params=pltpu.CompilerParams(needs_layout_passes=False),
    scratch_types=[
        pltpu.VMEM((BATCH,), jnp.int32),
        pltpu.VMEM((BATCH,), jnp.float32),
        pltpu.VMEM((EV_PER_W,), jnp.float32),
        pltpu.VMEM((EV_PER_W,), jnp.float32),
        pltpu.VMEM((R_OWN + 32,), jnp.float32),
        pltpu.SemaphoreType.DMA,
        pltpu.SemaphoreType.DMA,
        pltpu.SemaphoreType.DMA,
    ],
)(_body)


def kernel(node_ids, ts, last_src_ts):
    ids = node_ids.astype(jnp.int32)
    dt, new_last = _recency(ids, ts, last_src_ts)
    return dt, new_last
